# baseline (speedup 1.0000x reference)
import functools

import jax
import jax.numpy as jnp
from jax import lax
from jax.experimental import pallas as pl
from jax.experimental.pallas import tpu as pltpu
from jax.experimental.pallas import tpu_sc as plsc

_B, _T, _N, _E = 2, 30, 10000, 160000
_DIN, _DH, _DG = 8, 64, 128
_F = _B * _T
_NC, _NS = 2, 16
_FPC = _F // _NC
_EPT = _E // _NS
_BLK = 80
_NBLK = _EPT // _BLK
_RING = 8
_SCW = 4
_GAH = _RING - _SCW
_NPT = _N // _NS
_PN = 10240
_PNT = _PN // _NS
_BS = 2048
_NCHUNK = 2
_FC = _F // _NCHUNK
_FCC = _FC // _NC


def _sc_mesh():
    return plsc.VectorSubcoreMesh(core_axis_name="c", subcore_axis_name="s",
                                  num_cores=_NC, num_subcores=_NS)


@functools.cache
def _make_sc_degree():
    return functools.partial(
        pl.kernel,
        out_type=jax.ShapeDtypeStruct((_F, _PN), jnp.float32),
        mesh=_sc_mesh(),
        compiler_params=pltpu.CompilerParams(use_tc_tiling_on_sc=False),
        scratch_types=[
            pltpu.VMEM((_NBLK, _BLK), jnp.int32),
            pltpu.VMEM((_BLK,), jnp.float32),
            pltpu.VMEM((_PNT,), jnp.float32),
            pltpu.VMEM_SHARED((_PN,), jnp.float32),
            pltpu.SemaphoreType.DMA,
        ],
    )(_sc_degree_body)


def _sc_degree_body(ei_hbm, deg_hbm, dst_v, ones_v, zero_v, deg_sh, sem):
    c = lax.axis_index("c")
    s = lax.axis_index("s")

    for q in range(_BLK // 16):
        ones_v[pl.ds(q * 16, 16)] = jnp.ones((16,), jnp.float32)

    def _z(r, carry):
        zero_v[pl.ds(r * 16, 16)] = jnp.zeros((16,), jnp.float32)
        return carry

    lax.fori_loop(0, _PNT // 16, _z, 0)

    def _frame(i, carry):
        f = c * _FPC + i
        pltpu.sync_copy(ei_hbm.at[f, 1, s], dst_v)
        pltpu.sync_copy(zero_v, deg_sh.at[pl.ds(s * _PNT, _PNT)])
        plsc.subcore_barrier()

        def _fire(j, cc):
            pltpu.async_copy(ones_v, deg_sh.at[dst_v.at[j]], sem, add=True)
            return cc

        lax.fori_loop(0, _NBLK, _fire, 0)

        def _drain(j, cc):
            pltpu.make_async_copy(ones_v, deg_sh.at[dst_v.at[j]], sem).wait()
            return cc

        lax.fori_loop(0, _NBLK, _drain, 0)
        plsc.subcore_barrier()
        pltpu.sync_copy(deg_sh.at[pl.ds(s * _PNT, _PNT)],
                        deg_hbm.at[f, pl.ds(s * _PNT, _PNT)])
        plsc.subcore_barrier()
        return carry

    lax.fori_loop(0, _FPC, _frame, 0)


@functools.cache
def _make_sc_scatter(k):
    return functools.partial(
        pl.kernel,
        out_type=jax.ShapeDtypeStruct((_FC, _PN, _DH), jnp.float32),
        mesh=_sc_mesh(),
        compiler_params=pltpu.CompilerParams(use_tc_tiling_on_sc=False),
        scratch_types=[
            pltpu.VMEM((_NBLK, _BLK), jnp.int32),
            pltpu.VMEM((_NBLK, _BLK), jnp.int32),
            pltpu.VMEM((_RING, _BLK, _DH), jnp.float32),
            pltpu.VMEM((128, _DH), jnp.float32),
            pltpu.VMEM_SHARED((_PN, _DH), jnp.float32),
            pltpu.SemaphoreType.DMA,
            pltpu.SemaphoreType.DMA,
        ],
    )(functools.partial(_sc_scatter_body, k))


def _sc_scatter_body(k, y_hbm, ei_hbm, out_hbm, src_v, dst_v, ring_v, zero_v,
                     acc_sh, gsem, ssem):
    c = lax.axis_index("c")
    s = lax.axis_index("s")

    def _zrow(r, carry):
        for q in range(_DH // 16):
            zero_v[r, pl.ds(q * 16, 16)] = jnp.zeros((16,), jnp.float32)
        return carry

    lax.fori_loop(0, 128, _zrow, 0)

    for q in range(_PNT // 128):
        pltpu.sync_copy(zero_v, acc_sh.at[pl.ds(s * _PNT + q * 128, 128)])
    plsc.subcore_barrier()

    def _frame(i, carry):
        f = c * _FCC + i
        fe = k * _FC + f
        pltpu.sync_copy(ei_hbm.at[fe, 0, s], src_v)
        pltpu.sync_copy(ei_hbm.at[fe, 1, s], dst_v)

        for j in range(_GAH):
            pltpu.async_copy(y_hbm.at[f].at[src_v.at[j]], ring_v.at[j], gsem)

        def _blk(j, cc):
            slot = lax.rem(j, _RING)
            pltpu.make_async_copy(y_hbm.at[f].at[src_v.at[j]],
                                  ring_v.at[slot], gsem).wait()
            pltpu.async_copy(ring_v.at[slot], acc_sh.at[dst_v.at[j]], ssem,
                             add=True)

            @pl.when(j >= _SCW)
            def _():
                jd = j - _SCW
                pltpu.make_async_copy(ring_v.at[lax.rem(jd, _RING)],
                                      acc_sh.at[dst_v.at[jd]], ssem).wait()

            @pl.when(j + _GAH < _NBLK)
            def _():
                jg = j + _GAH
                pltpu.async_copy(y_hbm.at[f].at[src_v.at[jg]],
                                 ring_v.at[lax.rem(jg, _RING)], gsem)

            return cc

        lax.fori_loop(0, _NBLK, _blk, 0)

        def _tail(j, cc):
            pltpu.make_async_copy(ring_v.at[lax.rem(j, _RING)],
                                  acc_sh.at[dst_v.at[j]], ssem).wait()
            return cc

        lax.fori_loop(_NBLK - _SCW, _NBLK, _tail, 0)
        plsc.subcore_barrier()
        pltpu.sync_copy(acc_sh.at[pl.ds(s * _PNT, _PNT)],
                        out_hbm.at[f, pl.ds(s * _PNT, _PNT)])
        for q in range(_PNT // 128):
            pltpu.sync_copy(zero_v, acc_sh.at[pl.ds(s * _PNT + q * 128, 128)])
        plsc.subcore_barrier()
        return carry

    lax.fori_loop(0, _FCC, _frame, 0)


def _dot(a, b):
    return lax.dot_general(a, b, (((1,), (0,)), ((), ())),
                           precision=lax.Precision.HIGHEST,
                           preferred_element_type=jnp.float32)


def _tc1_body(x_ref, w1_ref, deg_ref, out_ref):
    dinv = lax.rsqrt(deg_ref[0] + 1.0)
    out_ref[0] = _dot(x_ref[0], w1_ref[...]) * dinv


def _tc2_body(s1_ref, y1_ref, deg_ref, b1_ref, w2_ref, out_ref):
    dinv = lax.rsqrt(deg_ref[0] + 1.0)
    h = jnp.maximum(dinv * (s1_ref[0] + y1_ref[0]) + b1_ref[...], 0.0)
    out_ref[0] = _dot(h, w2_ref[...]) * dinv


def _tc3_body(s2_ref, y2_ref, deg_ref, b2_ref, out_ref):
    nb = pl.program_id(1)
    dinv = lax.rsqrt(deg_ref[0] + 1.0)
    o = jnp.maximum(dinv * (s2_ref[0] + y2_ref[0]) + b2_ref[...], 0.0)
    rows = nb * _BS + lax.broadcasted_iota(jnp.int32, (_BS, 1), 0)
    o = jnp.where(rows < _N, o, 0.0)

    @pl.when(nb == 0)
    def _():
        out_ref[0] = jnp.zeros((1, _DH), jnp.float32)

    out_ref[0] += jnp.sum(o, axis=0, keepdims=True) * (1.0 / _N)


def _gru_body(embs_ref, wiT_ref, whT_ref, bih_ref, bhh_ref, wfT_ref, bf_ref,
              out_ref):
    wiT = wiT_ref[...]
    whT = whT_ref[...]
    bih = bih_ref[...]
    bhh = bhh_ref[...]

    def step(t, h):
        x0 = embs_ref[pl.ds(t, 1), :]
        x1 = embs_ref[pl.ds(t + _T, 1), :]
        xt = jnp.concatenate([x0, x1], axis=0)
        gi = _dot(xt, wiT) + bih
        gh = _dot(h, whT) + bhh
        r = jax.nn.sigmoid(gi[:, :_DG] + gh[:, :_DG])
        z = jax.nn.sigmoid(gi[:, _DG:2 * _DG] + gh[:, _DG:2 * _DG])
        n = jnp.tanh(gi[:, 2 * _DG:] + r * gh[:, 2 * _DG:])
        return (1.0 - z) * n + z * h

    h = lax.fori_loop(0, _T, step, jnp.zeros((_B, _DG), jnp.float32))
    out_ref[...] = _dot(h, wfT_ref[...]) + bf_ref[...]


def _make_tc_calls(k, interpret=False):
    grid = (_FC, _PN // _BS)
    off = k * _FC
    fdh = pl.BlockSpec((1, _BS, _DH), lambda f, nb: (f, nb, 0))
    fdeg = pl.BlockSpec((1, _BS, 1), lambda f, nb: (off + f, nb, 0))
    bias = pl.BlockSpec((1, _DH), lambda f, nb: (0, 0))
    tc1 = pl.pallas_call(
        _tc1_body, grid=grid,
        in_specs=[pl.BlockSpec((1, _BS, _DIN), lambda f, nb: (off + f, nb, 0)),
                  pl.BlockSpec((_DIN, _DH), lambda f, nb: (0, 0)), fdeg],
        out_specs=fdh,
        out_shape=jax.ShapeDtypeStruct((_FC, _PN, _DH), jnp.float32),
        interpret=interpret)
    tc2 = pl.pallas_call(
        _tc2_body, grid=grid,
        in_specs=[fdh, fdh, fdeg, bias,
                  pl.BlockSpec((_DH, _DH), lambda f, nb: (0, 0))],
        out_specs=fdh,
        out_shape=jax.ShapeDtypeStruct((_FC, _PN, _DH), jnp.float32),
        interpret=interpret)
    tc3 = pl.pallas_call(
        _tc3_body, grid=grid,
        in_specs=[fdh, fdh, fdeg, bias],
        out_specs=pl.BlockSpec((1, 1, _DH), lambda f, nb: (f, 0, 0)),
        out_shape=jax.ShapeDtypeStruct((_FC, 1, _DH), jnp.float32),
        interpret=interpret)
    return tc1, tc2, tc3


def _make_gru(interpret=False):
    return pl.pallas_call(
        _gru_body,
        out_shape=jax.ShapeDtypeStruct((_B, 2), jnp.float32),
        interpret=interpret)


def kernel(x, edge_index, W1, b1, W2, b2, W_ih, W_hh, b_ih, b_hh, Wf, bf):
    xs = x.reshape(_F, _N, _DIN)
    ei = edge_index.reshape(_F, 2, _NS, _NBLK, _BLK)
    deg = _make_sc_degree()(ei).reshape(_F, _PN, 1)
    embs = []
    for k in range(_NCHUNK):
        tc1, tc2, tc3 = _make_tc_calls(k)
        sc_scatter = _make_sc_scatter(k)
        y1 = tc1(xs, W1, deg)
        s1 = sc_scatter(y1, ei)
        y2 = tc2(s1, y1, deg, b1.reshape(1, _DH), W2)
        s2 = sc_scatter(y2, ei)
        embs.append(tc3(s2, y2, deg, b2.reshape(1, _DH)).reshape(_FC, _DH))
    embs = jnp.concatenate(embs, axis=0)
    return _make_gru()(embs, W_ih.T, W_hh.T, b_ih.reshape(1, 3 * _DG),
                       b_hh.reshape(1, 3 * _DG), Wf.T, bf.reshape(1, 2))


# speedup vs baseline: 1.0802x; 1.0802x over previous
import functools

import jax
import jax.numpy as jnp
from jax import lax
from jax.experimental import pallas as pl
from jax.experimental.pallas import tpu as pltpu
from jax.experimental.pallas import tpu_sc as plsc

_B, _T, _N, _E = 2, 30, 10000, 160000
_DIN, _DH, _DG = 8, 64, 128
_F = _B * _T
_NC, _NS = 2, 16
_FPC = _F // _NC
_EPT = _E // _NS
_BLK = 80
_NBLK = _EPT // _BLK
_RING = 8
_SCW = 4
_GAH = _RING - _SCW
_NPT = _N // _NS
_PN = 10240
_PNT = _PN // _NS
_BS = 2048
_NCHUNK = 6
_FC = _F // _NCHUNK
_FCC = _FC // _NC


def _sc_mesh():
    return plsc.VectorSubcoreMesh(core_axis_name="c", subcore_axis_name="s",
                                  num_cores=_NC, num_subcores=_NS)


@functools.cache
def _make_sc_degree():
    return functools.partial(
        pl.kernel,
        out_type=jax.ShapeDtypeStruct((_F, _PN), jnp.float32),
        mesh=_sc_mesh(),
        compiler_params=pltpu.CompilerParams(use_tc_tiling_on_sc=False),
        scratch_types=[
            pltpu.VMEM((_NBLK, _BLK), jnp.int32),
            pltpu.VMEM((_BLK,), jnp.float32),
            pltpu.VMEM((_PNT,), jnp.float32),
            pltpu.VMEM_SHARED((_PN,), jnp.float32),
            pltpu.SemaphoreType.DMA,
        ],
    )(_sc_degree_body)


def _sc_degree_body(ei_hbm, deg_hbm, dst_v, ones_v, zero_v, deg_sh, sem):
    c = lax.axis_index("c")
    s = lax.axis_index("s")

    for q in range(_BLK // 16):
        ones_v[pl.ds(q * 16, 16)] = jnp.ones((16,), jnp.float32)

    def _z(r, carry):
        zero_v[pl.ds(r * 16, 16)] = jnp.zeros((16,), jnp.float32)
        return carry

    lax.fori_loop(0, _PNT // 16, _z, 0)

    def _frame(i, carry):
        f = c * _FPC + i
        pltpu.sync_copy(ei_hbm.at[f, 1, s], dst_v)
        pltpu.sync_copy(zero_v, deg_sh.at[pl.ds(s * _PNT, _PNT)])
        plsc.subcore_barrier()

        def _fire(j, cc):
            pltpu.async_copy(ones_v, deg_sh.at[dst_v.at[j]], sem, add=True)
            return cc

        lax.fori_loop(0, _NBLK, _fire, 0)

        def _drain(j, cc):
            pltpu.make_async_copy(ones_v, deg_sh.at[dst_v.at[j]], sem).wait()
            return cc

        lax.fori_loop(0, _NBLK, _drain, 0)
        plsc.subcore_barrier()
        pltpu.sync_copy(deg_sh.at[pl.ds(s * _PNT, _PNT)],
                        deg_hbm.at[f, pl.ds(s * _PNT, _PNT)])
        plsc.subcore_barrier()
        return carry

    lax.fori_loop(0, _FPC, _frame, 0)


@functools.cache
def _make_sc_scatter(k):
    return functools.partial(
        pl.kernel,
        out_type=jax.ShapeDtypeStruct((_FC, _PN, _DH), jnp.float32),
        mesh=_sc_mesh(),
        compiler_params=pltpu.CompilerParams(use_tc_tiling_on_sc=False),
        scratch_types=[
            pltpu.VMEM((_NBLK, _BLK), jnp.int32),
            pltpu.VMEM((_NBLK, _BLK), jnp.int32),
            pltpu.VMEM((_RING, _BLK, _DH), jnp.float32),
            pltpu.VMEM((128, _DH), jnp.float32),
            pltpu.VMEM_SHARED((_PN, _DH), jnp.float32),
            pltpu.SemaphoreType.DMA,
            pltpu.SemaphoreType.DMA,
        ],
    )(functools.partial(_sc_scatter_body, k))


def _sc_scatter_body(k, y_hbm, ei_hbm, out_hbm, src_v, dst_v, ring_v, zero_v,
                     acc_sh, gsem, ssem):
    c = lax.axis_index("c")
    s = lax.axis_index("s")

    def _zrow(r, carry):
        for q in range(_DH // 16):
            zero_v[r, pl.ds(q * 16, 16)] = jnp.zeros((16,), jnp.float32)
        return carry

    lax.fori_loop(0, 128, _zrow, 0)

    for q in range(_PNT // 128):
        pltpu.sync_copy(zero_v, acc_sh.at[pl.ds(s * _PNT + q * 128, 128)])
    plsc.subcore_barrier()

    def _frame(i, carry):
        f = c * _FCC + i
        fe = k * _FC + f
        pltpu.sync_copy(ei_hbm.at[fe, 0, s], src_v)
        pltpu.sync_copy(ei_hbm.at[fe, 1, s], dst_v)

        for j in range(_GAH):
            pltpu.async_copy(y_hbm.at[f].at[src_v.at[j]], ring_v.at[j], gsem)

        def _blk(j, cc):
            slot = lax.rem(j, _RING)
            pltpu.make_async_copy(y_hbm.at[f].at[src_v.at[j]],
                                  ring_v.at[slot], gsem).wait()
            pltpu.async_copy(ring_v.at[slot], acc_sh.at[dst_v.at[j]], ssem,
                             add=True)

            @pl.when(j >= _SCW)
            def _():
                jd = j - _SCW
                pltpu.make_async_copy(ring_v.at[lax.rem(jd, _RING)],
                                      acc_sh.at[dst_v.at[jd]], ssem).wait()

            @pl.when(j + _GAH < _NBLK)
            def _():
                jg = j + _GAH
                pltpu.async_copy(y_hbm.at[f].at[src_v.at[jg]],
                                 ring_v.at[lax.rem(jg, _RING)], gsem)

            return cc

        lax.fori_loop(0, _NBLK, _blk, 0)

        def _tail(j, cc):
            pltpu.make_async_copy(ring_v.at[lax.rem(j, _RING)],
                                  acc_sh.at[dst_v.at[j]], ssem).wait()
            return cc

        lax.fori_loop(_NBLK - _SCW, _NBLK, _tail, 0)
        plsc.subcore_barrier()
        pltpu.sync_copy(acc_sh.at[pl.ds(s * _PNT, _PNT)],
                        out_hbm.at[f, pl.ds(s * _PNT, _PNT)])
        for q in range(_PNT // 128):
            pltpu.sync_copy(zero_v, acc_sh.at[pl.ds(s * _PNT + q * 128, 128)])
        plsc.subcore_barrier()
        return carry

    lax.fori_loop(0, _FCC, _frame, 0)


def _dot(a, b):
    return lax.dot_general(a, b, (((1,), (0,)), ((), ())),
                           precision=lax.Precision.HIGHEST,
                           preferred_element_type=jnp.float32)


def _tc1_body(x_ref, w1_ref, deg_ref, out_ref):
    dinv = lax.rsqrt(deg_ref[0] + 1.0)
    out_ref[0] = _dot(x_ref[0], w1_ref[...]) * dinv


def _tc2_body(s1_ref, y1_ref, deg_ref, b1_ref, w2_ref, out_ref):
    dinv = lax.rsqrt(deg_ref[0] + 1.0)
    h = jnp.maximum(dinv * (s1_ref[0] + y1_ref[0]) + b1_ref[...], 0.0)
    out_ref[0] = _dot(h, w2_ref[...]) * dinv


def _tc3_body(s2_ref, y2_ref, deg_ref, b2_ref, out_ref):
    nb = pl.program_id(1)
    dinv = lax.rsqrt(deg_ref[0] + 1.0)
    o = jnp.maximum(dinv * (s2_ref[0] + y2_ref[0]) + b2_ref[...], 0.0)
    rows = nb * _BS + lax.broadcasted_iota(jnp.int32, (_BS, 1), 0)
    o = jnp.where(rows < _N, o, 0.0)

    @pl.when(nb == 0)
    def _():
        out_ref[0] = jnp.zeros((1, _DH), jnp.float32)

    out_ref[0] += jnp.sum(o, axis=0, keepdims=True) * (1.0 / _N)


def _gru_body(embs_ref, wiT_ref, whT_ref, bih_ref, bhh_ref, wfT_ref, bf_ref,
              out_ref):
    wiT = wiT_ref[...]
    whT = whT_ref[...]
    bih = bih_ref[...]
    bhh = bhh_ref[...]

    def step(t, h):
        x0 = embs_ref[pl.ds(t, 1), :]
        x1 = embs_ref[pl.ds(t + _T, 1), :]
        xt = jnp.concatenate([x0, x1], axis=0)
        gi = _dot(xt, wiT) + bih
        gh = _dot(h, whT) + bhh
        r = jax.nn.sigmoid(gi[:, :_DG] + gh[:, :_DG])
        z = jax.nn.sigmoid(gi[:, _DG:2 * _DG] + gh[:, _DG:2 * _DG])
        n = jnp.tanh(gi[:, 2 * _DG:] + r * gh[:, 2 * _DG:])
        return (1.0 - z) * n + z * h

    h = lax.fori_loop(0, _T, step, jnp.zeros((_B, _DG), jnp.float32))
    out_ref[...] = _dot(h, wfT_ref[...]) + bf_ref[...]


def _make_tc_calls(k, interpret=False):
    grid = (_FC, _PN // _BS)
    off = k * _FC
    fdh = pl.BlockSpec((1, _BS, _DH), lambda f, nb: (f, nb, 0))
    fdeg = pl.BlockSpec((1, _BS, 1), lambda f, nb: (off + f, nb, 0))
    bias = pl.BlockSpec((1, _DH), lambda f, nb: (0, 0))
    tc1 = pl.pallas_call(
        _tc1_body, grid=grid,
        in_specs=[pl.BlockSpec((1, _BS, _DIN), lambda f, nb: (off + f, nb, 0)),
                  pl.BlockSpec((_DIN, _DH), lambda f, nb: (0, 0)), fdeg],
        out_specs=fdh,
        out_shape=jax.ShapeDtypeStruct((_FC, _PN, _DH), jnp.float32),
        interpret=interpret)
    tc2 = pl.pallas_call(
        _tc2_body, grid=grid,
        in_specs=[fdh, fdh, fdeg, bias,
                  pl.BlockSpec((_DH, _DH), lambda f, nb: (0, 0))],
        out_specs=fdh,
        out_shape=jax.ShapeDtypeStruct((_FC, _PN, _DH), jnp.float32),
        interpret=interpret)
    tc3 = pl.pallas_call(
        _tc3_body, grid=grid,
        in_specs=[fdh, fdh, fdeg, bias],
        out_specs=pl.BlockSpec((1, 1, _DH), lambda f, nb: (f, 0, 0)),
        out_shape=jax.ShapeDtypeStruct((_FC, 1, _DH), jnp.float32),
        interpret=interpret)
    return tc1, tc2, tc3


def _make_gru(interpret=False):
    return pl.pallas_call(
        _gru_body,
        out_shape=jax.ShapeDtypeStruct((_B, 2), jnp.float32),
        interpret=interpret)


def kernel(x, edge_index, W1, b1, W2, b2, W_ih, W_hh, b_ih, b_hh, Wf, bf):
    xs = x.reshape(_F, _N, _DIN)
    ei = edge_index.reshape(_F, 2, _NS, _NBLK, _BLK)
    deg = _make_sc_degree()(ei).reshape(_F, _PN, 1)
    embs = []
    for k in range(_NCHUNK):
        tc1, tc2, tc3 = _make_tc_calls(k)
        sc_scatter = _make_sc_scatter(k)
        y1 = tc1(xs, W1, deg)
        s1 = sc_scatter(y1, ei)
        y2 = tc2(s1, y1, deg, b1.reshape(1, _DH), W2)
        s2 = sc_scatter(y2, ei)
        embs.append(tc3(s2, y2, deg, b2.reshape(1, _DH)).reshape(_FC, _DH))
    embs = jnp.concatenate(embs, axis=0)
    return _make_gru()(embs, W_ih.T, W_hh.T, b_ih.reshape(1, 3 * _DG),
                       b_hh.reshape(1, 3 * _DG), Wf.T, bf.reshape(1, 2))


# speedup vs baseline: 1.0910x; 1.0100x over previous
import functools

import jax
import jax.numpy as jnp
from jax import lax
from jax.experimental import pallas as pl
from jax.experimental.pallas import tpu as pltpu
from jax.experimental.pallas import tpu_sc as plsc

_B, _T, _N, _E = 2, 30, 10000, 160000
_DIN, _DH, _DG = 8, 64, 128
_F = _B * _T
_NC, _NS = 2, 16
_FPC = _F // _NC
_EPT = _E // _NS
_BLK = 80
_NBLK = _EPT // _BLK
_RING = 8
_SCW = 4
_GAH = _RING - _SCW
_NPT = _N // _NS
_PN = 10240
_PNT = _PN // _NS
_BS = 2048
_NCHUNK = 10
_FC = _F // _NCHUNK
_FCC = _FC // _NC


def _sc_mesh():
    return plsc.VectorSubcoreMesh(core_axis_name="c", subcore_axis_name="s",
                                  num_cores=_NC, num_subcores=_NS)


@functools.cache
def _make_sc_degree():
    return functools.partial(
        pl.kernel,
        out_type=jax.ShapeDtypeStruct((_F, _PN), jnp.float32),
        mesh=_sc_mesh(),
        compiler_params=pltpu.CompilerParams(use_tc_tiling_on_sc=False),
        scratch_types=[
            pltpu.VMEM((_NBLK, _BLK), jnp.int32),
            pltpu.VMEM((_BLK,), jnp.float32),
            pltpu.VMEM((_PNT,), jnp.float32),
            pltpu.VMEM_SHARED((_PN,), jnp.float32),
            pltpu.SemaphoreType.DMA,
        ],
    )(_sc_degree_body)


def _sc_degree_body(ei_hbm, deg_hbm, dst_v, ones_v, zero_v, deg_sh, sem):
    c = lax.axis_index("c")
    s = lax.axis_index("s")

    for q in range(_BLK // 16):
        ones_v[pl.ds(q * 16, 16)] = jnp.ones((16,), jnp.float32)

    def _z(r, carry):
        zero_v[pl.ds(r * 16, 16)] = jnp.zeros((16,), jnp.float32)
        return carry

    lax.fori_loop(0, _PNT // 16, _z, 0)

    def _frame(i, carry):
        f = c * _FPC + i
        pltpu.sync_copy(ei_hbm.at[f, 1, s], dst_v)
        pltpu.sync_copy(zero_v, deg_sh.at[pl.ds(s * _PNT, _PNT)])
        plsc.subcore_barrier()

        def _fire(j, cc):
            pltpu.async_copy(ones_v, deg_sh.at[dst_v.at[j]], sem, add=True)
            return cc

        lax.fori_loop(0, _NBLK, _fire, 0)

        def _drain(j, cc):
            pltpu.make_async_copy(ones_v, deg_sh.at[dst_v.at[j]], sem).wait()
            return cc

        lax.fori_loop(0, _NBLK, _drain, 0)
        plsc.subcore_barrier()
        pltpu.sync_copy(deg_sh.at[pl.ds(s * _PNT, _PNT)],
                        deg_hbm.at[f, pl.ds(s * _PNT, _PNT)])
        plsc.subcore_barrier()
        return carry

    lax.fori_loop(0, _FPC, _frame, 0)


@functools.cache
def _make_sc_scatter(k):
    return functools.partial(
        pl.kernel,
        out_type=jax.ShapeDtypeStruct((_FC, _PN, _DH), jnp.float32),
        mesh=_sc_mesh(),
        compiler_params=pltpu.CompilerParams(use_tc_tiling_on_sc=False),
        scratch_types=[
            pltpu.VMEM((_NBLK, _BLK), jnp.int32),
            pltpu.VMEM((_NBLK, _BLK), jnp.int32),
            pltpu.VMEM((_RING, _BLK, _DH), jnp.float32),
            pltpu.VMEM((128, _DH), jnp.float32),
            pltpu.VMEM_SHARED((_PN, _DH), jnp.float32),
            pltpu.SemaphoreType.DMA,
            pltpu.SemaphoreType.DMA,
        ],
    )(functools.partial(_sc_scatter_body, k))


def _sc_scatter_body(k, y_hbm, ei_hbm, out_hbm, src_v, dst_v, ring_v, zero_v,
                     acc_sh, gsem, ssem):
    c = lax.axis_index("c")
    s = lax.axis_index("s")

    def _zrow(r, carry):
        for q in range(_DH // 16):
            zero_v[r, pl.ds(q * 16, 16)] = jnp.zeros((16,), jnp.float32)
        return carry

    lax.fori_loop(0, 128, _zrow, 0)

    for q in range(_PNT // 128):
        pltpu.sync_copy(zero_v, acc_sh.at[pl.ds(s * _PNT + q * 128, 128)])
    plsc.subcore_barrier()

    def _frame(i, carry):
        f = c * _FCC + i
        fe = k * _FC + f
        pltpu.sync_copy(ei_hbm.at[fe, 0, s], src_v)
        pltpu.sync_copy(ei_hbm.at[fe, 1, s], dst_v)

        for j in range(_GAH):
            pltpu.async_copy(y_hbm.at[f].at[src_v.at[j]], ring_v.at[j], gsem)

        def _blk(j, cc):
            slot = lax.rem(j, _RING)
            pltpu.make_async_copy(y_hbm.at[f].at[src_v.at[j]],
                                  ring_v.at[slot], gsem).wait()
            pltpu.async_copy(ring_v.at[slot], acc_sh.at[dst_v.at[j]], ssem,
                             add=True)

            @pl.when(j >= _SCW)
            def _():
                jd = j - _SCW
                pltpu.make_async_copy(ring_v.at[lax.rem(jd, _RING)],
                                      acc_sh.at[dst_v.at[jd]], ssem).wait()

            @pl.when(j + _GAH < _NBLK)
            def _():
                jg = j + _GAH
                pltpu.async_copy(y_hbm.at[f].at[src_v.at[jg]],
                                 ring_v.at[lax.rem(jg, _RING)], gsem)

            return cc

        lax.fori_loop(0, _NBLK, _blk, 0)

        def _tail(j, cc):
            pltpu.make_async_copy(ring_v.at[lax.rem(j, _RING)],
                                  acc_sh.at[dst_v.at[j]], ssem).wait()
            return cc

        lax.fori_loop(_NBLK - _SCW, _NBLK, _tail, 0)
        plsc.subcore_barrier()
        pltpu.sync_copy(acc_sh.at[pl.ds(s * _PNT, _PNT)],
                        out_hbm.at[f, pl.ds(s * _PNT, _PNT)])
        for q in range(_PNT // 128):
            pltpu.sync_copy(zero_v, acc_sh.at[pl.ds(s * _PNT + q * 128, 128)])
        plsc.subcore_barrier()
        return carry

    lax.fori_loop(0, _FCC, _frame, 0)


def _dot(a, b):
    return lax.dot_general(a, b, (((1,), (0,)), ((), ())),
                           precision=lax.Precision.HIGHEST,
                           preferred_element_type=jnp.float32)


def _tc1_body(x_ref, w1_ref, deg_ref, out_ref):
    dinv = lax.rsqrt(deg_ref[0] + 1.0)
    out_ref[0] = _dot(x_ref[0], w1_ref[...]) * dinv


def _tc2_body(s1_ref, y1_ref, deg_ref, b1_ref, w2_ref, out_ref):
    dinv = lax.rsqrt(deg_ref[0] + 1.0)
    h = jnp.maximum(dinv * (s1_ref[0] + y1_ref[0]) + b1_ref[...], 0.0)
    out_ref[0] = _dot(h, w2_ref[...]) * dinv


def _tc3_body(s2_ref, y2_ref, deg_ref, b2_ref, out_ref):
    nb = pl.program_id(1)
    dinv = lax.rsqrt(deg_ref[0] + 1.0)
    o = jnp.maximum(dinv * (s2_ref[0] + y2_ref[0]) + b2_ref[...], 0.0)
    rows = nb * _BS + lax.broadcasted_iota(jnp.int32, (_BS, 1), 0)
    o = jnp.where(rows < _N, o, 0.0)

    @pl.when(nb == 0)
    def _():
        out_ref[0] = jnp.zeros((1, _DH), jnp.float32)

    out_ref[0] += jnp.sum(o, axis=0, keepdims=True) * (1.0 / _N)


def _gru_body(embs_ref, wiT_ref, whT_ref, bih_ref, bhh_ref, wfT_ref, bf_ref,
              out_ref):
    wiT = wiT_ref[...]
    whT = whT_ref[...]
    bih = bih_ref[...]
    bhh = bhh_ref[...]

    def step(t, h):
        x0 = embs_ref[pl.ds(t, 1), :]
        x1 = embs_ref[pl.ds(t + _T, 1), :]
        xt = jnp.concatenate([x0, x1], axis=0)
        gi = _dot(xt, wiT) + bih
        gh = _dot(h, whT) + bhh
        r = jax.nn.sigmoid(gi[:, :_DG] + gh[:, :_DG])
        z = jax.nn.sigmoid(gi[:, _DG:2 * _DG] + gh[:, _DG:2 * _DG])
        n = jnp.tanh(gi[:, 2 * _DG:] + r * gh[:, 2 * _DG:])
        return (1.0 - z) * n + z * h

    h = lax.fori_loop(0, _T, step, jnp.zeros((_B, _DG), jnp.float32))
    out_ref[...] = _dot(h, wfT_ref[...]) + bf_ref[...]


def _make_tc_calls(k, interpret=False):
    grid = (_FC, _PN // _BS)
    off = k * _FC
    fdh = pl.BlockSpec((1, _BS, _DH), lambda f, nb: (f, nb, 0))
    fdeg = pl.BlockSpec((1, _BS, 1), lambda f, nb: (off + f, nb, 0))
    bias = pl.BlockSpec((1, _DH), lambda f, nb: (0, 0))
    tc1 = pl.pallas_call(
        _tc1_body, grid=grid,
        in_specs=[pl.BlockSpec((1, _BS, _DIN), lambda f, nb: (off + f, nb, 0)),
                  pl.BlockSpec((_DIN, _DH), lambda f, nb: (0, 0)), fdeg],
        out_specs=fdh,
        out_shape=jax.ShapeDtypeStruct((_FC, _PN, _DH), jnp.float32),
        interpret=interpret)
    tc2 = pl.pallas_call(
        _tc2_body, grid=grid,
        in_specs=[fdh, fdh, fdeg, bias,
                  pl.BlockSpec((_DH, _DH), lambda f, nb: (0, 0))],
        out_specs=fdh,
        out_shape=jax.ShapeDtypeStruct((_FC, _PN, _DH), jnp.float32),
        interpret=interpret)
    tc3 = pl.pallas_call(
        _tc3_body, grid=grid,
        in_specs=[fdh, fdh, fdeg, bias],
        out_specs=pl.BlockSpec((1, 1, _DH), lambda f, nb: (f, 0, 0)),
        out_shape=jax.ShapeDtypeStruct((_FC, 1, _DH), jnp.float32),
        interpret=interpret)
    return tc1, tc2, tc3


def _make_gru(interpret=False):
    return pl.pallas_call(
        _gru_body,
        out_shape=jax.ShapeDtypeStruct((_B, 2), jnp.float32),
        interpret=interpret)


def kernel(x, edge_index, W1, b1, W2, b2, W_ih, W_hh, b_ih, b_hh, Wf, bf):
    xs = x.reshape(_F, _N, _DIN)
    ei = edge_index.reshape(_F, 2, _NS, _NBLK, _BLK)
    deg = _make_sc_degree()(ei).reshape(_F, _PN, 1)
    embs = []
    for k in range(_NCHUNK):
        tc1, tc2, tc3 = _make_tc_calls(k)
        sc_scatter = _make_sc_scatter(k)
        y1 = tc1(xs, W1, deg)
        s1 = sc_scatter(y1, ei)
        y2 = tc2(s1, y1, deg, b1.reshape(1, _DH), W2)
        s2 = sc_scatter(y2, ei)
        embs.append(tc3(s2, y2, deg, b2.reshape(1, _DH)).reshape(_FC, _DH))
    embs = jnp.concatenate(embs, axis=0)
    return _make_gru()(embs, W_ih.T, W_hh.T, b_ih.reshape(1, 3 * _DG),
                       b_hh.reshape(1, 3 * _DG), Wf.T, bf.reshape(1, 2))
